# Initial kernel scaffold; baseline (speedup 1.0000x reference)
import jax
import jax.numpy as jnp
from jax.experimental import pallas as pl


def kernel(x):
    raise NotImplementedError("write your pallas kernel here")


# speedup vs baseline: 10.0744x; 10.0744x over previous
import functools

import jax
import jax.numpy as jnp
from jax import lax
from jax.experimental import pallas as pl
from jax.experimental.pallas import tpu as pltpu
from jax.experimental.pallas import tpu_sc as plsc

B = 64
N = 32768
K = 2048
L = 16
NV = N // L
NW = 32
ROWS_PER_W = B // NW

W0, W1, W2 = 10, 11, 10
NB0, NB1, NB2 = 1 << W0, 1 << W1, 1 << W2
HIST_MAX = NB1 * L

UNROLL = 8


def _row_topk(row_v, hist_v, tot_v, lane):
    def hist_pass(shift, width, prefix):
        nb = 1 << width

        def zero_body(j, _):
            base = j * (L * UNROLL)
            for u in range(UNROLL):
                hist_v[pl.ds(base + u * L, L)] = jnp.zeros((L,), jnp.int32)
            return 0

        lax.fori_loop(0, (nb * L) // (L * UNROLL), zero_body, 0)

        lane_off = lane * nb
        ones = jnp.ones((L,), jnp.int32)

        def body(i, _):
            base = i * (L * UNROLL)
            for u in range(UNROLL):
                v = row_v[pl.ds(base + u * L, L)]
                bits = lax.bitcast_convert_type(v, jnp.int32)
                bits = jnp.maximum(bits, 0)
                d = lax.shift_right_logical(bits, shift) & (nb - 1)
                if prefix is None:
                    plsc.addupdate_scatter(hist_v, [lane_off + d], ones)
                else:
                    m = lax.shift_right_logical(bits, shift + width) == prefix
                    plsc.addupdate_scatter(hist_v, [lane_off + d], ones, mask=m)
            return 0

        lax.fori_loop(0, NV // UNROLL, body, 0)

        def red_body(c, _):
            acc = hist_v[pl.ds(c * L, L)]
            for l in range(1, L):
                acc = acc + hist_v[pl.ds(l * nb + c * L, L)]
            tot_v[pl.ds(c * L, L)] = acc
            return 0

        lax.fori_loop(0, nb // L, red_body, 0)

    def find_bucket(width, k_rem):
        nch = (1 << width) // L

        def body(i, carry):
            cnt, above = carry
            c = nch - 1 - i
            v = tot_v[pl.ds(c * L, L)]
            suf = lax.rev(plsc.cumsum(lax.rev(v, (0,))), (0,)) + above
            cnt = cnt + jnp.sum(jnp.where(suf >= k_rem, 1, 0))
            above = above + jnp.sum(v)
            return cnt, above

        cnt, _ = lax.fori_loop(0, nch, body, (jnp.int32(0), jnp.int32(0)))
        b0 = cnt - 1

        def body2(c, acc):
            v = tot_v[pl.ds(c * L, L)]
            g = c * L + lane
            return acc + jnp.sum(jnp.where(g > b0, v, 0))

        s_above = lax.fori_loop(0, nch, body2, jnp.int32(0))
        return b0, s_above

    k0 = jnp.int32(K)
    hist_pass(W1 + W2, W0, None)
    b0, s0 = find_bucket(W0, k0)
    k1 = k0 - s0
    hist_pass(W2, W1, b0)
    b1, s1 = find_bucket(W1, k1)
    k2 = k1 - s1
    prefix1 = b0 * NB1 + b1
    hist_pass(0, W2, prefix1)
    b2, _ = find_bucket(W2, k2)
    tbits = prefix1 * NB2 + b2
    t = lax.bitcast_convert_type(tbits, jnp.float32)

    def out_body(i, _):
        base = i * (L * UNROLL)
        for u in range(UNROLL):
            v = row_v[pl.ds(base + u * L, L)]
            row_v[pl.ds(base + u * L, L)] = jnp.where(v >= t, v, 0.0)
        return 0

    lax.fori_loop(0, NV // UNROLL, out_body, 0)


@functools.partial(
    pl.kernel,
    out_type=jax.ShapeDtypeStruct((B, N), jnp.float32),
    mesh=plsc.VectorSubcoreMesh(core_axis_name="c", subcore_axis_name="s"),
    compiler_params=pltpu.CompilerParams(needs_layout_passes=False),
    scratch_types=[
        pltpu.VMEM((N,), jnp.float32),
        pltpu.VMEM((HIST_MAX,), jnp.int32),
        pltpu.VMEM((NB1,), jnp.int32),
    ],
)
def _topk_mask_sc(x_hbm, out_hbm, row_v, hist_v, tot_v):
    wid = lax.axis_index("s") * 2 + lax.axis_index("c")
    lane = lax.iota(jnp.int32, L)
    for r in range(ROWS_PER_W):
        row = wid * ROWS_PER_W + r
        pltpu.sync_copy(x_hbm.at[row], row_v)
        _row_topk(row_v, hist_v, tot_v, lane)
        pltpu.sync_copy(row_v, out_hbm.at[row])


def kernel(x):
    return _topk_mask_sc(x)


# speedup vs baseline: 22.6549x; 2.2488x over previous
import functools

import jax
import jax.numpy as jnp
from jax import lax
from jax.experimental import pallas as pl
from jax.experimental.pallas import tpu as pltpu
from jax.experimental.pallas import tpu_sc as plsc

B = 64
N = 32768
K = 2048
L = 16
NV = N // L
NW = 32
ROWS_PER_W = B // NW

W0, W1, W2 = 10, 11, 10
NB0, NB1, NB2 = 1 << W0, 1 << W1, 1 << W2
HIST_MAX = NB1 * L

UNROLL = 8


def _row_topk(row_v, hist_v, tot_v, lane):
    def hist_pass(shift, width, prefix):
        nb = 1 << width
        zeros = jnp.zeros((L,), jnp.int32)

        @plsc.parallel_loop(0, (nb * L) // L, unroll=UNROLL)
        def _(j):
            hist_v[pl.ds(j * L, L)] = zeros

        lane_off = lane * nb
        ones = jnp.ones((L,), jnp.int32)

        @plsc.parallel_loop(0, NV, unroll=UNROLL)
        def _(i):
            v = row_v[pl.ds(i * L, L)]
            bits = lax.bitcast_convert_type(v, jnp.int32)
            bits = jnp.maximum(bits, 0)
            d = lax.shift_right_logical(bits, shift) & (nb - 1)
            if prefix is None:
                plsc.addupdate_scatter(hist_v, [lane_off + d], ones)
            else:
                m = lax.shift_right_logical(bits, shift + width) == prefix
                plsc.addupdate_scatter(hist_v, [lane_off + d], ones, mask=m)

        @plsc.parallel_loop(0, nb // L, unroll=2)
        def _(c):
            acc = hist_v[pl.ds(c * L, L)]
            for l in range(1, L):
                acc = acc + hist_v[pl.ds(l * nb + c * L, L)]
            tot_v[pl.ds(c * L, L)] = acc

    def find_bucket(width, k_rem):
        nch = (1 << width) // L

        def body(i, carry):
            cnt, above = carry
            c = nch - 1 - i
            v = tot_v[pl.ds(c * L, L)]
            suf = lax.rev(plsc.cumsum(lax.rev(v, (0,))), (0,)) + above
            cnt = cnt + jnp.sum(jnp.where(suf >= k_rem, 1, 0))
            above = above + jnp.sum(v)
            return cnt, above

        cnt, _ = lax.fori_loop(0, nch, body, (jnp.int32(0), jnp.int32(0)))
        b0 = cnt - 1

        def body2(c, acc):
            v = tot_v[pl.ds(c * L, L)]
            g = c * L + lane
            return acc + jnp.sum(jnp.where(g > b0, v, 0))

        s_above = lax.fori_loop(0, nch, body2, jnp.int32(0))
        return b0, s_above

    k0 = jnp.int32(K)
    hist_pass(W1 + W2, W0, None)
    b0, s0 = find_bucket(W0, k0)
    k1 = k0 - s0
    hist_pass(W2, W1, b0)
    b1, s1 = find_bucket(W1, k1)
    k2 = k1 - s1
    prefix1 = b0 * NB1 + b1
    hist_pass(0, W2, prefix1)
    b2, _ = find_bucket(W2, k2)
    tbits = prefix1 * NB2 + b2
    t = lax.bitcast_convert_type(tbits, jnp.float32)

    @plsc.parallel_loop(0, NV, unroll=UNROLL)
    def _(i):
        v = row_v[pl.ds(i * L, L)]
        row_v[pl.ds(i * L, L)] = jnp.where(v >= t, v, 0.0)


@functools.partial(
    pl.kernel,
    out_type=jax.ShapeDtypeStruct((B, N), jnp.float32),
    mesh=plsc.VectorSubcoreMesh(core_axis_name="c", subcore_axis_name="s"),
    compiler_params=pltpu.CompilerParams(needs_layout_passes=False),
    scratch_types=[
        pltpu.VMEM((N,), jnp.float32),
        pltpu.VMEM((HIST_MAX,), jnp.int32),
        pltpu.VMEM((NB1,), jnp.int32),
    ],
)
def _topk_mask_sc(x_hbm, out_hbm, row_v, hist_v, tot_v):
    wid = lax.axis_index("s") * 2 + lax.axis_index("c")
    lane = lax.iota(jnp.int32, L)
    for r in range(ROWS_PER_W):
        row = wid * ROWS_PER_W + r
        pltpu.sync_copy(x_hbm.at[row], row_v)
        _row_topk(row_v, hist_v, tot_v, lane)
        pltpu.sync_copy(row_v, out_hbm.at[row])


def kernel(x):
    return _topk_mask_sc(x)


# speedup vs baseline: 31.0176x; 1.3691x over previous
import functools

import jax
import jax.numpy as jnp
from jax import lax
from jax.experimental import pallas as pl
from jax.experimental.pallas import tpu as pltpu
from jax.experimental.pallas import tpu_sc as plsc

B = 64
N = 32768
K = 2048
L = 16
NV = N // L
NW = 32
ROWS_PER_W = B // NW

W0, W1, W2 = 10, 11, 10
NB0, NB1, NB2 = 1 << W0, 1 << W1, 1 << W2
HIST_MAX = NB1 * L

INT_MAX = 2**31 - 1


def _row_topk(row_v, hist_v, tot_v, csum_v, lane):
    def hist_pass(shift, width, prefix):
        nb = 1 << width
        zeros = jnp.zeros((L,), jnp.int32)

        @plsc.parallel_loop(0, nb, unroll=8)
        def _(j):
            hist_v[pl.ds(j * L, L)] = zeros

        lane_off = lane * nb
        ones = jnp.ones((L,), jnp.int32)
        base = 0 if prefix is None else prefix * nb

        @plsc.parallel_loop(0, NV, unroll=16)
        def _(i):
            v = row_v[pl.ds(i * L, L)]
            bits = lax.bitcast_convert_type(v, jnp.uint32)
            d = (lax.shift_right_logical(bits, jnp.uint32(shift))
                 - jnp.uint32(base)).astype(jnp.int32)
            m = d.astype(jnp.uint32) < jnp.uint32(nb)
            plsc.addupdate_scatter(hist_v, [lane_off + d], ones, mask=m)

        @plsc.parallel_loop(0, nb // L, unroll=4)
        def _(c):
            acc = hist_v[pl.ds(c * L, L)]
            for l in range(1, L):
                acc = acc + hist_v[pl.ds(l * nb + c * L, L)]
            tot_v[pl.ds(c * L, L)] = acc
            s = jnp.sum(acc)
            cvec = jnp.full((L,), c, jnp.int32)
            svec = jnp.full((L,), s, jnp.int32)
            plsc.store_scatter(csum_v, [cvec], svec, mask=lane == 0)

    def find_bucket(width, k_rem):
        nch = (1 << width) // L
        nchv = nch // L

        def body(i, carry):
            cnt, above = carry
            cv = nchv - 1 - i
            v = csum_v[pl.ds(cv * L, L)]
            suf = lax.rev(plsc.cumsum(lax.rev(v, (0,))), (0,)) + above
            cnt = cnt + jnp.sum(jnp.where(suf >= k_rem, 1, 0))
            above = above + jnp.sum(v)
            return cnt, above

        cnt, _ = lax.fori_loop(0, nchv, body, (jnp.int32(0), jnp.int32(0)))
        c0 = cnt - 1

        def body2(cv, acc):
            v = csum_v[pl.ds(cv * L, L)]
            g = cv * L + lane
            return acc + jnp.sum(jnp.where(g > c0, v, 0))

        above_c0 = lax.fori_loop(0, nchv, body2, jnp.int32(0))

        v = tot_v[pl.ds(c0 * L, L)]
        suf = lax.rev(plsc.cumsum(lax.rev(v, (0,))), (0,)) + above_c0
        m = suf >= k_rem
        pc = jnp.sum(jnp.where(m, 1, 0))
        b0 = c0 * L + pc - 1
        s_b0 = jnp.min(jnp.where(m, suf, INT_MAX))
        v_b0 = jnp.sum(jnp.where(lane == pc - 1, v, 0))
        return b0, k_rem - (s_b0 - v_b0)

    k0 = jnp.int32(K)
    hist_pass(W1 + W2, W0, None)
    b0, k1 = find_bucket(W0, k0)
    hist_pass(W2, W1, b0)
    b1, k2 = find_bucket(W1, k1)
    prefix1 = b0 * NB1 + b1
    hist_pass(0, W2, prefix1)
    b2, _ = find_bucket(W2, k2)
    tbits = prefix1 * NB2 + b2
    t = lax.bitcast_convert_type(tbits, jnp.float32)

    @plsc.parallel_loop(0, NV, unroll=16)
    def _(i):
        v = row_v[pl.ds(i * L, L)]
        row_v[pl.ds(i * L, L)] = jnp.where(v >= t, v, 0.0)


@functools.partial(
    pl.kernel,
    out_type=jax.ShapeDtypeStruct((B, N), jnp.float32),
    mesh=plsc.VectorSubcoreMesh(core_axis_name="c", subcore_axis_name="s"),
    compiler_params=pltpu.CompilerParams(needs_layout_passes=False),
    scratch_types=[
        pltpu.VMEM((N,), jnp.float32),
        pltpu.VMEM((N,), jnp.float32),
        pltpu.VMEM((HIST_MAX,), jnp.int32),
        pltpu.VMEM((NB1,), jnp.int32),
        pltpu.VMEM((NB1 // L,), jnp.int32),
        pltpu.SemaphoreType.DMA,
        pltpu.SemaphoreType.DMA,
    ],
)
def _topk_mask_sc(x_hbm, out_hbm, row_a, row_b, hist_v, tot_v, csum_v,
                  sem_in, sem_out):
    wid = lax.axis_index("s") * 2 + lax.axis_index("c")
    lane = lax.iota(jnp.int32, L)
    row0 = wid * ROWS_PER_W
    row1 = row0 + 1

    pltpu.sync_copy(x_hbm.at[row0], row_a)
    cp_in1 = pltpu.async_copy(x_hbm.at[row1], row_b, sem_in)
    _row_topk(row_a, hist_v, tot_v, csum_v, lane)
    cp_out0 = pltpu.async_copy(row_a, out_hbm.at[row0], sem_out)
    cp_in1.wait()
    _row_topk(row_b, hist_v, tot_v, csum_v, lane)
    pltpu.sync_copy(row_b, out_hbm.at[row1])
    cp_out0.wait()


def kernel(x):
    return _topk_mask_sc(x)


# speedup vs baseline: 32.9579x; 1.0626x over previous
import functools

import jax
import jax.numpy as jnp
from jax import lax
from jax.experimental import pallas as pl
from jax.experimental.pallas import tpu as pltpu
from jax.experimental.pallas import tpu_sc as plsc

B = 64
N = 32768
K = 2048
L = 16
NV = N // L
NW = 32
ROWS_PER_W = B // NW

W0, W1, W2 = 11, 10, 10
NB0, NB1, NB2 = 1 << W0, 1 << W1, 1 << W2
STRIDE = NB0
HIST_WORDS = STRIDE * L

INT_MAX = 2**31 - 1


def _row_topk(row_v, hist_v, tot_v, csum_v, lane):
    def hist_pass(shift, width, prefix):
        nb = 1 << width
        lane_off = lane * STRIDE
        ones = jnp.ones((L,), jnp.int32)
        base = 0 if prefix is None else prefix * nb

        @plsc.parallel_loop(0, NV, unroll=16)
        def _(i):
            v = row_v[pl.ds(i * L, L)]
            bits = lax.bitcast_convert_type(v, jnp.uint32)
            d = (lax.shift_right_logical(bits, jnp.uint32(shift))
                 - jnp.uint32(base)).astype(jnp.int32)
            m = d.astype(jnp.uint32) < jnp.uint32(nb)
            plsc.addupdate_scatter(hist_v, [lane_off + d], ones, mask=m)

        zeros = jnp.zeros((L,), jnp.int32)

        @plsc.parallel_loop(0, nb // L, unroll=4)
        def _(c):
            acc = hist_v[pl.ds(c * L, L)]
            hist_v[pl.ds(c * L, L)] = zeros
            for l in range(1, L):
                acc = acc + hist_v[pl.ds(l * STRIDE + c * L, L)]
                hist_v[pl.ds(l * STRIDE + c * L, L)] = zeros
            tot_v[pl.ds(c * L, L)] = acc
            s = jnp.sum(acc)
            cvec = jnp.full((L,), c, jnp.int32)
            svec = jnp.full((L,), s, jnp.int32)
            plsc.store_scatter(csum_v, [cvec], svec, mask=lane == 0)

    def find_bucket(width, k_rem):
        nch = (1 << width) // L
        nchv = nch // L

        def body(i, carry):
            cnt, above = carry
            cv = nchv - 1 - i
            v = csum_v[pl.ds(cv * L, L)]
            suf = lax.rev(plsc.cumsum(lax.rev(v, (0,))), (0,)) + above
            cnt = cnt + jnp.sum(jnp.where(suf >= k_rem, 1, 0))
            above = above + jnp.sum(v)
            return cnt, above

        cnt, _ = lax.fori_loop(0, nchv, body, (jnp.int32(0), jnp.int32(0)))
        c0 = cnt - 1

        def body2(cv, acc):
            v = csum_v[pl.ds(cv * L, L)]
            g = cv * L + lane
            return acc + jnp.sum(jnp.where(g > c0, v, 0))

        above_c0 = lax.fori_loop(0, nchv, body2, jnp.int32(0))

        v = tot_v[pl.ds(c0 * L, L)]
        suf = lax.rev(plsc.cumsum(lax.rev(v, (0,))), (0,)) + above_c0
        m = suf >= k_rem
        pc = jnp.sum(jnp.where(m, 1, 0))
        b0 = c0 * L + pc - 1
        s_b0 = jnp.min(jnp.where(m, suf, INT_MAX))
        v_b0 = jnp.sum(jnp.where(lane == pc - 1, v, 0))
        return b0, k_rem - (s_b0 - v_b0)

    k0 = jnp.int32(K)
    hist_pass(W1 + W2, W0, None)
    b0, k1 = find_bucket(W0, k0)
    hist_pass(W2, W1, b0)
    b1, k2 = find_bucket(W1, k1)
    prefix1 = b0 * NB1 + b1
    hist_pass(0, W2, prefix1)
    b2, _ = find_bucket(W2, k2)
    tbits = prefix1 * NB2 + b2
    t = lax.bitcast_convert_type(tbits, jnp.float32)

    @plsc.parallel_loop(0, NV, unroll=16)
    def _(i):
        v = row_v[pl.ds(i * L, L)]
        row_v[pl.ds(i * L, L)] = jnp.where(v >= t, v, 0.0)


@functools.partial(
    pl.kernel,
    out_type=jax.ShapeDtypeStruct((B, N), jnp.float32),
    mesh=plsc.VectorSubcoreMesh(core_axis_name="c", subcore_axis_name="s"),
    compiler_params=pltpu.CompilerParams(needs_layout_passes=False),
    scratch_types=[
        pltpu.VMEM((N,), jnp.float32),
        pltpu.VMEM((N,), jnp.float32),
        pltpu.VMEM((HIST_WORDS,), jnp.int32),
        pltpu.VMEM((NB0,), jnp.int32),
        pltpu.VMEM((NB0 // L,), jnp.int32),
        pltpu.SemaphoreType.DMA,
        pltpu.SemaphoreType.DMA,
        pltpu.SemaphoreType.DMA,
    ],
)
def _topk_mask_sc(x_hbm, out_hbm, row_a, row_b, hist_v, tot_v, csum_v,
                  sem_in0, sem_in1, sem_out):
    wid = lax.axis_index("s") * 2 + lax.axis_index("c")
    lane = lax.iota(jnp.int32, L)
    row0 = wid * ROWS_PER_W
    row1 = row0 + 1

    cp_in0 = pltpu.async_copy(x_hbm.at[row0], row_a, sem_in0)
    cp_in1 = pltpu.async_copy(x_hbm.at[row1], row_b, sem_in1)

    zeros = jnp.zeros((L,), jnp.int32)

    @plsc.parallel_loop(0, HIST_WORDS // L, unroll=8)
    def _(j):
        hist_v[pl.ds(j * L, L)] = zeros

    cp_in0.wait()
    _row_topk(row_a, hist_v, tot_v, csum_v, lane)
    cp_out0 = pltpu.async_copy(row_a, out_hbm.at[row0], sem_out)
    cp_in1.wait()
    _row_topk(row_b, hist_v, tot_v, csum_v, lane)
    pltpu.sync_copy(row_b, out_hbm.at[row1])
    cp_out0.wait()


def kernel(x):
    return _topk_mask_sc(x)
